# baseline (speedup 1.0000x reference)
import jax
import jax.numpy as jnp
from jax import lax
from jax.experimental import pallas as pl
from jax.experimental.pallas import tpu as pltpu
from jax.experimental.pallas import tpu_sc as plsc

NC, NS, L = 2, 16, 16
NW = NC * NS
B = 16384
D = 50
NOUT = 5
BPW = B // NW
GRP = BPW // L
BLK = 4
SLAB = BLK * L
DP = 64
NROW = BPW * BLK
IPC = 128
EPC = IPC // BLK
NIDX = NROW // IPC
Q = 4
NITER = GRP // Q
WB_ROWS = 256
B_OFF = D * NOUT


def _sc_body(user_h, product_h, uf_h, pf_h, wb_h, out_h,
             raw_u, raw_p, bidx_u, bidx_p,
             rows_u, rows_p, wb_v, out_v, sem):
    wid = lax.axis_index("s") * NC + lax.axis_index("c")
    base = wid * BPW

    pltpu.sync_copy(user_h.at[pl.ds(base, BPW)], raw_u)
    pltpu.sync_copy(product_h.at[pl.ds(base, BPW)], raw_p)
    pltpu.sync_copy(wb_h, wb_v)

    def bld(g, c):
        row = jnp.full((L,), 0, jnp.int32) + (g >> 1)
        cb = ((g & 1) << 6) + lax.iota(jnp.int32, L) * BLK
        for raw, bidx in ((raw_u, bidx_u), (raw_p, bidx_p)):
            blk = (raw[pl.ds(g * L, L)] * DP) >> 4
            for c2 in range(BLK):
                plsc.store_scatter(bidx, [row, cb + c2], blk + c2)
        return c

    lax.fori_loop(0, GRP, bld, 0)

    copies = []
    for k in range(NIDX):
        copies.append(pltpu.async_copy(
            uf_h.at[bidx_u.at[k]], rows_u.at[pl.ds(k * IPC, IPC)], sem))
        copies.append(pltpu.async_copy(
            pf_h.at[bidx_p.at[k]], rows_p.at[pl.ds(k * IPC, IPC)], sem))
    for cp in copies:
        cp.wait()

    def iter_body(i, carry):
        accs = [[wb_v[B_OFF + j] for j in range(NOUT)] for _ in range(Q)]
        obase_u, obase_p = [], []
        for sub in range(Q):
            p0 = i * (Q * L) + sub * L
            slab0 = (p0 + lax.iota(jnp.int32, L)) * SLAB
            obase_u.append(slab0)
            obase_p.append(slab0)
        for f in range(D):
            wvecs = [wb_v[j * D + f] for j in range(NOUT)]
            for sub in range(Q):
                tu = obase_u[sub] + f
                tp = obase_p[sub] + f
                u = plsc.load_gather(rows_u, [tu >> 4, tu & 15])
                p = plsc.load_gather(rows_p, [tp >> 4, tp & 15])
                inter = u * p
                for j in range(NOUT):
                    accs[sub][j] = accs[sub][j] + inter * wvecs[j]
        for sub in range(Q):
            obase = (i * (Q * L) + sub * L + lax.iota(jnp.int32, L)) * NOUT
            for j in range(NOUT):
                plsc.store_scatter(out_v, [obase + j], accs[sub][j])
        return carry

    lax.fori_loop(0, NITER, iter_body, 0)
    pltpu.sync_copy(out_v, out_h.at[pl.ds(base * NOUT, BPW * NOUT)])


@jax.jit
def kernel(user, product, user_factors, product_factors, W, b):
    uf16 = jnp.pad(user_factors, ((0, 0), (0, DP - D))).reshape(-1, L)
    pf16 = jnp.pad(product_factors, ((0, 0), (0, DP - D))).reshape(-1, L)
    wb = jnp.concatenate([
        W.astype(jnp.float32).reshape(D * NOUT, 1),
        b.astype(jnp.float32).reshape(NOUT, 1),
        jnp.zeros((WB_ROWS - D * NOUT - NOUT, 1), jnp.float32),
    ], axis=0)
    wb = jnp.broadcast_to(wb, (WB_ROWS, L))

    mesh = plsc.VectorSubcoreMesh(core_axis_name="c", subcore_axis_name="s",
                                  num_cores=NC, num_subcores=NS)
    fn = pl.kernel(
        _sc_body,
        out_type=jax.ShapeDtypeStruct((B * NOUT,), jnp.float32),
        mesh=mesh,
        compiler_params=pltpu.CompilerParams(needs_layout_passes=False,
                                             use_tc_tiling_on_sc=False),
        scratch_types=[
            pltpu.VMEM((BPW,), jnp.int32),
            pltpu.VMEM((BPW,), jnp.int32),
            pltpu.VMEM((NIDX, IPC), jnp.int32),
            pltpu.VMEM((NIDX, IPC), jnp.int32),
            pltpu.VMEM((NROW, L), jnp.float32),
            pltpu.VMEM((NROW, L), jnp.float32),
            pltpu.VMEM((WB_ROWS, L), jnp.float32),
            pltpu.VMEM((BPW * NOUT,), jnp.float32),
            pltpu.SemaphoreType.DMA,
        ],
    )
    flat = fn(user.astype(jnp.int32), product.astype(jnp.int32),
              uf16, pf16, wb)
    return flat.reshape(B, NOUT)


# speedup vs baseline: 1.0828x; 1.0828x over previous
import jax
import jax.numpy as jnp
from jax import lax
from jax.experimental import pallas as pl
from jax.experimental.pallas import tpu as pltpu
from jax.experimental.pallas import tpu_sc as plsc

NC, NS, L = 2, 16, 16
NW = NC * NS
B = 16384
D = 50
NOUT = 5
BPW = B // NW
GRP = BPW // L
BLK = 4
SLAB = BLK * L
NROW = BPW * BLK
IPC = 128
EPC = IPC // BLK
NIDX = NROW // IPC
Q = 4
NITER = GRP // Q
WB_ROWS = 256
B_OFF = D * NOUT


def _sc_body(user_h, product_h, uf_h, pf_h, wb_h, out_h,
             raw_u, raw_p, bidx_u, bidx_p,
             rows_u, rows_p, wb_v, out_v, sem):
    wid = lax.axis_index("s") * NC + lax.axis_index("c")
    base = wid * BPW

    pltpu.sync_copy(user_h.at[pl.ds(base, BPW)], raw_u)
    pltpu.sync_copy(product_h.at[pl.ds(base, BPW)], raw_p)
    pltpu.sync_copy(wb_h, wb_v)

    def bld(g, c):
        row = jnp.full((L,), 0, jnp.int32) + (g >> 1)
        cb = ((g & 1) << 6) + lax.iota(jnp.int32, L) * BLK
        for raw, bidx in ((raw_u, bidx_u), (raw_p, bidx_p)):
            blk = (raw[pl.ds(g * L, L)] * D) >> 4
            for c2 in range(BLK):
                plsc.store_scatter(bidx, [row, cb + c2], blk + c2)
        return c

    lax.fori_loop(0, GRP, bld, 0)

    copies = []
    for k in range(NIDX):
        copies.append(pltpu.async_copy(
            uf_h.at[bidx_u.at[k]], rows_u.at[pl.ds(k * IPC, IPC)], sem))
        copies.append(pltpu.async_copy(
            pf_h.at[bidx_p.at[k]], rows_p.at[pl.ds(k * IPC, IPC)], sem))
    for cp in copies:
        cp.wait()

    def iter_body(i, carry):
        accs = [[wb_v[B_OFF + j] for j in range(NOUT)] for _ in range(Q)]
        obase_u, obase_p = [], []
        for sub in range(Q):
            p0 = i * (Q * L) + sub * L
            slab0 = (p0 + lax.iota(jnp.int32, L)) * SLAB
            obase_u.append(slab0 + ((raw_u[pl.ds(p0, L)] * D) & 15))
            obase_p.append(slab0 + ((raw_p[pl.ds(p0, L)] * D) & 15))
        for f in range(D):
            wvecs = [wb_v[j * D + f] for j in range(NOUT)]
            for sub in range(Q):
                tu = obase_u[sub] + f
                tp = obase_p[sub] + f
                u = plsc.load_gather(rows_u, [tu >> 4, tu & 15])
                p = plsc.load_gather(rows_p, [tp >> 4, tp & 15])
                inter = u * p
                for j in range(NOUT):
                    accs[sub][j] = accs[sub][j] + inter * wvecs[j]
        for sub in range(Q):
            evec = i * (Q * L) + sub * L + lax.iota(jnp.int32, L)
            jz = jnp.full((L,), 0, jnp.int32)
            for j in range(NOUT):
                plsc.store_scatter(out_v, [evec, jz + j], accs[sub][j])
        return carry

    lax.fori_loop(0, NITER, iter_body, 0)
    pltpu.sync_copy(out_v, out_h.at[pl.ds(base, BPW)])


@jax.jit
def kernel(user, product, user_factors, product_factors, W, b):
    uf16 = user_factors.reshape(-1, L)
    pf16 = product_factors.reshape(-1, L)
    wb = jnp.concatenate([
        W.astype(jnp.float32).reshape(D * NOUT, 1),
        b.astype(jnp.float32).reshape(NOUT, 1),
        jnp.zeros((WB_ROWS - D * NOUT - NOUT, 1), jnp.float32),
    ], axis=0)
    wb = jnp.broadcast_to(wb, (WB_ROWS, L))

    mesh = plsc.VectorSubcoreMesh(core_axis_name="c", subcore_axis_name="s",
                                  num_cores=NC, num_subcores=NS)
    fn = pl.kernel(
        _sc_body,
        out_type=jax.ShapeDtypeStruct((B, NOUT), jnp.float32),
        mesh=mesh,
        compiler_params=pltpu.CompilerParams(needs_layout_passes=False,
                                             use_tc_tiling_on_sc=False),
        scratch_types=[
            pltpu.VMEM((BPW,), jnp.int32),
            pltpu.VMEM((BPW,), jnp.int32),
            pltpu.VMEM((NIDX, IPC), jnp.int32),
            pltpu.VMEM((NIDX, IPC), jnp.int32),
            pltpu.VMEM((NROW, L), jnp.float32),
            pltpu.VMEM((NROW, L), jnp.float32),
            pltpu.VMEM((WB_ROWS, L), jnp.float32),
            pltpu.VMEM((BPW, NOUT), jnp.float32),
            pltpu.SemaphoreType.DMA,
        ],
    )
    return fn(user.astype(jnp.int32), product.astype(jnp.int32),
              uf16, pf16, wb)
